# Initial kernel scaffold; baseline (speedup 1.0000x reference)
import jax
import jax.numpy as jnp
from jax.experimental import pallas as pl


def kernel(x, w):
    raise NotImplementedError("write your pallas kernel here")


# speedup vs baseline: 1.3493x; 1.3493x over previous
import functools

import jax
import jax.numpy as jnp
from jax import lax
from jax.experimental import pallas as pl
from jax.experimental.pallas import tpu as pltpu
from jax.experimental.pallas import tpu_sc as plsc

EMB_DIM = 32
NUM_CODES = 8192
HALF = NUM_CODES // 2
TOK_BLOCK = 256


def _half_min(distances, col0):
    m = jnp.min(distances, axis=1, keepdims=True)
    col = col0 + lax.broadcasted_iota(jnp.int32, distances.shape, 1)
    idx = jnp.min(jnp.where(distances == m, col, NUM_CODES), axis=1)
    return m, idx


def _distance_argmin_body(xbf_ref, wbf_ref, wsq_ref, znorm_ref,
                          idx_ref, loss_ref, *, n_total):
    i = pl.program_id(0)
    dots = jnp.dot(xbf_ref[...], wbf_ref[...],
                   preferred_element_type=jnp.float32)
    znorm = znorm_ref[...].reshape(-1, 1)
    distances = (znorm - 2.0 * dots) + wsq_ref[...]
    mA, iA = _half_min(distances[:, :HALF], 0)
    mB, iB = _half_min(distances[:, HALF:], HALF)
    neg_a_bf = (-mA).astype(jnp.bfloat16).astype(jnp.float32)
    take_b = (-mB) > neg_a_bf
    idx_ref[...] = jnp.where(take_b[:, 0], iB, iA).astype(jnp.int32)
    m = jnp.where(take_b, mB, mA)

    @pl.when(i == 0)
    def _():
        loss_ref[...] = jnp.zeros((1, 1), jnp.float32)

    loss_ref[...] += jnp.sum(m).reshape(1, 1) * (2.0 / n_total)


def _distance_argmin(xbf, wbf, wsq, znorm):
    n_tok = xbf.shape[0]
    grid = n_tok // TOK_BLOCK
    return pl.pallas_call(
        functools.partial(_distance_argmin_body, n_total=n_tok * EMB_DIM),
        grid=(grid,),
        in_specs=[
            pl.BlockSpec((TOK_BLOCK, EMB_DIM), lambda i: (i, 0)),
            pl.BlockSpec((EMB_DIM, NUM_CODES), lambda i: (0, 0)),
            pl.BlockSpec((1, NUM_CODES), lambda i: (0, 0)),
            pl.BlockSpec((TOK_BLOCK,), lambda i: (i,)),
        ],
        out_specs=[
            pl.BlockSpec((TOK_BLOCK,), lambda i: (i,)),
            pl.BlockSpec((1, 1), lambda i: (0, 0)),
        ],
        out_shape=[
            jax.ShapeDtypeStruct((n_tok,), jnp.int32),
            jax.ShapeDtypeStruct((1, 1), jnp.float32),
        ],
        compiler_params=pltpu.CompilerParams(
            dimension_semantics=("arbitrary",),
        ),
    )(xbf, wbf, wsq, znorm)


def _make_sc_gather(n_tok):
    info = plsc.get_sparse_core_info()
    nc, ns = info.num_cores, info.num_subcores
    nw = nc * ns
    b_per_w = n_tok // nw
    chunks = b_per_w // 128

    @functools.partial(
        pl.kernel,
        mesh=plsc.VectorSubcoreMesh(core_axis_name="c", subcore_axis_name="s"),
        out_type=jax.ShapeDtypeStruct((n_tok, EMB_DIM), jnp.float32),
        scratch_types=[
            pltpu.VMEM((chunks, 128), jnp.int32),
            pltpu.VMEM((b_per_w, EMB_DIM), jnp.float32),
            pltpu.SemaphoreType.DMA,
        ],
        compiler_params=pltpu.CompilerParams(use_tc_tiling_on_sc=False),
    )
    def gather_kernel(table_hbm, idx_hbm, out_hbm, idx_v, rows_v, sem):
        wid = lax.axis_index("s") * nc + lax.axis_index("c")
        pltpu.sync_copy(idx_hbm.at[pl.ds(wid * chunks, chunks)], idx_v)
        copies = [
            pltpu.async_copy(
                table_hbm.at[idx_v.at[j]],
                rows_v.at[pl.ds(j * 128, 128)],
                sem,
            )
            for j in range(chunks)
        ]
        for c in copies:
            c.wait()
        pltpu.sync_copy(rows_v, out_hbm.at[pl.ds(wid * b_per_w, b_per_w)])

    return gather_kernel


def kernel(x, w):
    flat_x = x.reshape(-1, EMB_DIM)
    n_tok = flat_x.shape[0]
    xbf = flat_x.astype(jnp.bfloat16)
    wbf = w.astype(jnp.bfloat16)
    wsq = jnp.sum(w ** 2, axis=0, keepdims=True)
    znorm = jnp.sum(flat_x ** 2, axis=1)
    idx_flat, loss = _distance_argmin(xbf, wbf, wsq, znorm)
    table = w.T
    idx_rows = idx_flat.reshape(-1, 128)
    zq_flat = _make_sc_gather(n_tok)(table, idx_rows)
    encoding_indices = idx_flat.reshape(x.shape[:-1])
    z_q = zq_flat.reshape(x.shape)
    latent_loss = loss[0, 0]
    return (encoding_indices, z_q, x, latent_loss)


# speedup vs baseline: 1.4274x; 1.0579x over previous
import functools

import jax
import jax.numpy as jnp
from jax import lax
from jax.experimental import pallas as pl
from jax.experimental.pallas import tpu as pltpu
from jax.experimental.pallas import tpu_sc as plsc

EMB_DIM = 32
NUM_CODES = 8192
HALF = NUM_CODES // 2
TOK_BLOCK = 512


def _half_min(distances, col0):
    m = jnp.min(distances, axis=1, keepdims=True)
    col = col0 + lax.broadcasted_iota(jnp.int32, distances.shape, 1)
    idx = jnp.min(jnp.where(distances == m, col, NUM_CODES), axis=1)
    return m, idx


def _distance_argmin_body(xbf_ref, wbf_ref, wsq_ref, znorm_ref,
                          idx_ref, loss_ref, *, n_total):
    i = pl.program_id(0)
    dots = jnp.dot(xbf_ref[...], wbf_ref[...],
                   preferred_element_type=jnp.float32)
    znorm = znorm_ref[...].reshape(-1, 1)
    distances = (znorm - 2.0 * dots) + wsq_ref[...]
    mA, iA = _half_min(distances[:, :HALF], 0)
    mB, iB = _half_min(distances[:, HALF:], HALF)
    neg_a_bf = (-mA).astype(jnp.bfloat16).astype(jnp.float32)
    take_b = (-mB) > neg_a_bf
    idx_ref[...] = jnp.where(take_b[:, 0], iB, iA).astype(jnp.int32)
    m = jnp.where(take_b, mB, mA)

    @pl.when(i == 0)
    def _():
        loss_ref[...] = jnp.zeros((1, 1), jnp.float32)

    loss_ref[...] += jnp.sum(m).reshape(1, 1) * (2.0 / n_total)


def _distance_argmin(xbf, wbf, wsq, znorm):
    n_tok = xbf.shape[0]
    grid = n_tok // TOK_BLOCK
    return pl.pallas_call(
        functools.partial(_distance_argmin_body, n_total=n_tok * EMB_DIM),
        grid=(grid,),
        in_specs=[
            pl.BlockSpec((TOK_BLOCK, EMB_DIM), lambda i: (i, 0)),
            pl.BlockSpec((EMB_DIM, NUM_CODES), lambda i: (0, 0)),
            pl.BlockSpec((1, NUM_CODES), lambda i: (0, 0)),
            pl.BlockSpec((TOK_BLOCK,), lambda i: (i,)),
        ],
        out_specs=[
            pl.BlockSpec((TOK_BLOCK,), lambda i: (i,)),
            pl.BlockSpec((1, 1), lambda i: (0, 0)),
        ],
        out_shape=[
            jax.ShapeDtypeStruct((n_tok,), jnp.int32),
            jax.ShapeDtypeStruct((1, 1), jnp.float32),
        ],
        compiler_params=pltpu.CompilerParams(
            dimension_semantics=("arbitrary",),
        ),
    )(xbf, wbf, wsq, znorm)


def _make_sc_gather(n_tok):
    info = plsc.get_sparse_core_info()
    nc, ns = info.num_cores, info.num_subcores
    nw = nc * ns
    b_per_w = n_tok // nw
    chunks = b_per_w // 128

    @functools.partial(
        pl.kernel,
        mesh=plsc.VectorSubcoreMesh(core_axis_name="c", subcore_axis_name="s"),
        out_type=jax.ShapeDtypeStruct((n_tok, EMB_DIM), jnp.float32),
        scratch_types=[
            pltpu.VMEM((chunks, 128), jnp.int32),
            pltpu.VMEM((b_per_w, EMB_DIM), jnp.float32),
            pltpu.SemaphoreType.DMA,
        ],
        compiler_params=pltpu.CompilerParams(use_tc_tiling_on_sc=False),
    )
    def gather_kernel(table_hbm, idx_hbm, out_hbm, idx_v, rows_v, sem):
        wid = lax.axis_index("s") * nc + lax.axis_index("c")
        pltpu.sync_copy(idx_hbm.at[pl.ds(wid * chunks, chunks)], idx_v)
        copies = [
            pltpu.async_copy(
                table_hbm.at[idx_v.at[j]],
                rows_v.at[pl.ds(j * 128, 128)],
                sem,
            )
            for j in range(chunks)
        ]
        for c in copies:
            c.wait()
        pltpu.sync_copy(rows_v, out_hbm.at[pl.ds(wid * b_per_w, b_per_w)])

    return gather_kernel


def kernel(x, w):
    flat_x = x.reshape(-1, EMB_DIM)
    n_tok = flat_x.shape[0]
    xbf = flat_x.astype(jnp.bfloat16)
    wbf = w.astype(jnp.bfloat16)
    wsq = jnp.sum(w ** 2, axis=0, keepdims=True)
    znorm = jnp.sum(flat_x ** 2, axis=1)
    idx_flat, loss = _distance_argmin(xbf, wbf, wsq, znorm)
    table = w.T
    idx_rows = idx_flat.reshape(-1, 128)
    zq_flat = _make_sc_gather(n_tok)(table, idx_rows)
    encoding_indices = idx_flat.reshape(x.shape[:-1])
    z_q = zq_flat.reshape(x.shape)
    latent_loss = loss[0, 0]
    return (encoding_indices, z_q, x, latent_loss)


# speedup vs baseline: 1.4541x; 1.0187x over previous
import functools

import jax
import jax.numpy as jnp
from jax import lax
from jax.experimental import pallas as pl
from jax.experimental.pallas import tpu as pltpu
from jax.experimental.pallas import tpu_sc as plsc

EMB_DIM = 32
NUM_CODES = 8192
HALF = NUM_CODES // 2
TOK_BLOCK = 1024


def _half_min(distances, col0):
    m = jnp.min(distances, axis=1, keepdims=True)
    col = col0 + lax.broadcasted_iota(jnp.int32, distances.shape, 1)
    idx = jnp.min(jnp.where(distances == m, col, NUM_CODES), axis=1)
    return m, idx


def _distance_argmin_body(xbf_ref, wbf_ref, wsq_ref, znorm_ref,
                          idx_ref, loss_ref, *, n_total):
    i = pl.program_id(0)
    dots = jnp.dot(xbf_ref[...], wbf_ref[...],
                   preferred_element_type=jnp.float32)
    znorm = znorm_ref[...].reshape(-1, 1)
    distances = (znorm - 2.0 * dots) + wsq_ref[...]
    mA, iA = _half_min(distances[:, :HALF], 0)
    mB, iB = _half_min(distances[:, HALF:], HALF)
    neg_a_bf = (-mA).astype(jnp.bfloat16).astype(jnp.float32)
    take_b = (-mB) > neg_a_bf
    idx_ref[...] = jnp.where(take_b[:, 0], iB, iA).astype(jnp.int32)
    m = jnp.where(take_b, mB, mA)

    @pl.when(i == 0)
    def _():
        loss_ref[...] = jnp.zeros((1, 1), jnp.float32)

    loss_ref[...] += jnp.sum(m).reshape(1, 1) * (2.0 / n_total)


def _distance_argmin(xbf, wbf, wsq, znorm):
    n_tok = xbf.shape[0]
    grid = n_tok // TOK_BLOCK
    return pl.pallas_call(
        functools.partial(_distance_argmin_body, n_total=n_tok * EMB_DIM),
        grid=(grid,),
        in_specs=[
            pl.BlockSpec((TOK_BLOCK, EMB_DIM), lambda i: (i, 0)),
            pl.BlockSpec((EMB_DIM, NUM_CODES), lambda i: (0, 0)),
            pl.BlockSpec((1, NUM_CODES), lambda i: (0, 0)),
            pl.BlockSpec((TOK_BLOCK,), lambda i: (i,)),
        ],
        out_specs=[
            pl.BlockSpec((TOK_BLOCK,), lambda i: (i,)),
            pl.BlockSpec((1, 1), lambda i: (0, 0)),
        ],
        out_shape=[
            jax.ShapeDtypeStruct((n_tok,), jnp.int32),
            jax.ShapeDtypeStruct((1, 1), jnp.float32),
        ],
        compiler_params=pltpu.CompilerParams(
            dimension_semantics=("arbitrary",),
            vmem_limit_bytes=100 * 1024 * 1024,
        ),
    )(xbf, wbf, wsq, znorm)


def _make_sc_gather(n_tok):
    info = plsc.get_sparse_core_info()
    nc, ns = info.num_cores, info.num_subcores
    nw = nc * ns
    b_per_w = n_tok // nw
    chunks = b_per_w // 128

    @functools.partial(
        pl.kernel,
        mesh=plsc.VectorSubcoreMesh(core_axis_name="c", subcore_axis_name="s"),
        out_type=jax.ShapeDtypeStruct((n_tok, EMB_DIM), jnp.float32),
        scratch_types=[
            pltpu.VMEM((chunks, 128), jnp.int32),
            pltpu.VMEM((b_per_w, EMB_DIM), jnp.float32),
            pltpu.SemaphoreType.DMA,
        ],
        compiler_params=pltpu.CompilerParams(use_tc_tiling_on_sc=False),
    )
    def gather_kernel(table_hbm, idx_hbm, out_hbm, idx_v, rows_v, sem):
        wid = lax.axis_index("s") * nc + lax.axis_index("c")
        pltpu.sync_copy(idx_hbm.at[pl.ds(wid * chunks, chunks)], idx_v)
        copies = [
            pltpu.async_copy(
                table_hbm.at[idx_v.at[j]],
                rows_v.at[pl.ds(j * 128, 128)],
                sem,
            )
            for j in range(chunks)
        ]
        for c in copies:
            c.wait()
        pltpu.sync_copy(rows_v, out_hbm.at[pl.ds(wid * b_per_w, b_per_w)])

    return gather_kernel


def kernel(x, w):
    flat_x = x.reshape(-1, EMB_DIM)
    n_tok = flat_x.shape[0]
    xbf = flat_x.astype(jnp.bfloat16)
    wbf = w.astype(jnp.bfloat16)
    wsq = jnp.sum(w ** 2, axis=0, keepdims=True)
    znorm = jnp.sum(flat_x ** 2, axis=1)
    idx_flat, loss = _distance_argmin(xbf, wbf, wsq, znorm)
    table = w.T
    idx_rows = idx_flat.reshape(-1, 128)
    zq_flat = _make_sc_gather(n_tok)(table, idx_rows)
    encoding_indices = idx_flat.reshape(x.shape[:-1])
    z_q = zq_flat.reshape(x.shape)
    latent_loss = loss[0, 0]
    return (encoding_indices, z_q, x, latent_loss)


# speedup vs baseline: 1.5742x; 1.0826x over previous
import functools

import jax
import jax.numpy as jnp
from jax import lax
from jax.experimental import pallas as pl
from jax.experimental.pallas import tpu as pltpu
from jax.experimental.pallas import tpu_sc as plsc

EMB_DIM = 32
NUM_CODES = 8192
HALF = NUM_CODES // 2
TOK_BLOCK = 1024


def _half_min(distances, colv):
    m = jnp.min(distances, axis=1, keepdims=True)
    idx = jnp.min(jnp.where(distances == m, colv, float(NUM_CODES)), axis=1)
    return m, idx


def _distance_argmin_body(xbf_ref, wbf_ref, wsq_ref, znorm_ref, col_ref,
                          idx_ref, loss_ref, *, n_total):
    i = pl.program_id(0)
    dots2 = jnp.dot(xbf_ref[...], wbf_ref[...],
                    preferred_element_type=jnp.float32)
    znorm = znorm_ref[...].reshape(-1, 1)
    distances = (znorm - dots2) + wsq_ref[...]
    colv = col_ref[...]
    mA, iA = _half_min(distances[:, :HALF], colv)
    mB, iB = _half_min(distances[:, HALF:], colv)
    iB = iB + float(HALF)
    neg_a_bf = (-mA).astype(jnp.bfloat16).astype(jnp.float32)
    take_b = (-mB) > neg_a_bf
    idx_ref[...] = jnp.where(take_b[:, 0], iB, iA).astype(jnp.int32)
    m = jnp.where(take_b, mB, mA)

    @pl.when(i == 0)
    def _():
        loss_ref[...] = jnp.zeros((1, 1), jnp.float32)

    loss_ref[...] += jnp.sum(m).reshape(1, 1) * (2.0 / n_total)


def _distance_argmin(xbf, wbf, wsq, znorm, colv):
    n_tok = xbf.shape[0]
    grid = n_tok // TOK_BLOCK
    return pl.pallas_call(
        functools.partial(_distance_argmin_body, n_total=n_tok * EMB_DIM),
        grid=(grid,),
        in_specs=[
            pl.BlockSpec((TOK_BLOCK, EMB_DIM), lambda i: (i, 0)),
            pl.BlockSpec((EMB_DIM, NUM_CODES), lambda i: (0, 0)),
            pl.BlockSpec((1, NUM_CODES), lambda i: (0, 0)),
            pl.BlockSpec((TOK_BLOCK,), lambda i: (i,)),
            pl.BlockSpec((1, HALF), lambda i: (0, 0)),
        ],
        out_specs=[
            pl.BlockSpec((TOK_BLOCK,), lambda i: (i,)),
            pl.BlockSpec((1, 1), lambda i: (0, 0)),
        ],
        out_shape=[
            jax.ShapeDtypeStruct((n_tok,), jnp.int32),
            jax.ShapeDtypeStruct((1, 1), jnp.float32),
        ],
        compiler_params=pltpu.CompilerParams(
            dimension_semantics=("arbitrary",),
            vmem_limit_bytes=100 * 1024 * 1024,
        ),
    )(xbf, wbf, wsq, znorm, colv)


def _make_sc_gather(n_tok):
    info = plsc.get_sparse_core_info()
    nc, ns = info.num_cores, info.num_subcores
    nw = nc * ns
    b_per_w = n_tok // nw
    chunks = b_per_w // 128

    @functools.partial(
        pl.kernel,
        mesh=plsc.VectorSubcoreMesh(core_axis_name="c", subcore_axis_name="s"),
        out_type=jax.ShapeDtypeStruct((n_tok, EMB_DIM), jnp.float32),
        scratch_types=[
            pltpu.VMEM((chunks, 128), jnp.int32),
            pltpu.VMEM((b_per_w, EMB_DIM), jnp.float32),
            pltpu.SemaphoreType.DMA,
        ],
        compiler_params=pltpu.CompilerParams(use_tc_tiling_on_sc=False),
    )
    def gather_kernel(table_hbm, idx_hbm, out_hbm, idx_v, rows_v, sem):
        wid = lax.axis_index("s") * nc + lax.axis_index("c")
        pltpu.sync_copy(idx_hbm.at[pl.ds(wid * chunks, chunks)], idx_v)
        copies = [
            pltpu.async_copy(
                table_hbm.at[idx_v.at[j]],
                rows_v.at[pl.ds(j * 128, 128)],
                sem,
            )
            for j in range(chunks)
        ]
        for c in copies:
            c.wait()
        pltpu.sync_copy(rows_v, out_hbm.at[pl.ds(wid * b_per_w, b_per_w)])

    return gather_kernel


def kernel(x, w):
    flat_x = x.reshape(-1, EMB_DIM)
    n_tok = flat_x.shape[0]
    xbf = (2.0 * flat_x).astype(jnp.bfloat16)
    wbf = w.astype(jnp.bfloat16)
    wsq = jnp.sum(w ** 2, axis=0, keepdims=True)
    znorm = jnp.sum(flat_x ** 2, axis=1)
    colv = lax.broadcasted_iota(jnp.float32, (1, HALF), 1)
    idx_flat, loss = _distance_argmin(xbf, wbf, wsq, znorm, colv)
    table = w.T
    idx_rows = idx_flat.reshape(-1, 128)
    zq_flat = _make_sc_gather(n_tok)(table, idx_rows)
    encoding_indices = idx_flat.reshape(x.shape[:-1])
    z_q = zq_flat.reshape(x.shape)
    latent_loss = loss[0, 0]
    return (encoding_indices, z_q, x, latent_loss)
